# Initial kernel scaffold; baseline (speedup 1.0000x reference)
import jax
import jax.numpy as jnp
from jax.experimental import pallas as pl


def kernel(at_no, pos, edge_index, fc_edge_index, embed_table, W_filt, b_filt, W_self, W_gate, W_up1, W_up2, Wn1, Wn2, We1, We2, Wg0, Wnode_out, Wp, Wedge_out):
    raise NotImplementedError("write your pallas kernel here")


# speedup vs baseline: 10.5854x; 10.5854x over previous
import functools
import jax
import jax.numpy as jnp
from jax import lax
from jax.experimental import pallas as pl

N = 1024
E = 16384
EF = 65536
C = 128
NB = 32
H = 128
S = 32
B = 16
NL = 4
NA = 2
CUTOFF = 5.0

BE = 2048
BF = 2048

_bf16 = jnp.bfloat16
_f32 = jnp.float32
_HI = lax.Precision.HIGHEST


def _sigmoid(x):
    return 1.0 / (1.0 + jnp.exp(-x))


def _silu(x):
    return x * _sigmoid(x)


def _dot_ref(a, b):
    return jnp.dot(a.astype(_bf16), b.astype(_bf16), preferred_element_type=_f32)


def _onehot_T(idx2d, rows, cols, dtype):
    return (lax.broadcasted_iota(jnp.int32, (rows, cols), 0) == idx2d).astype(dtype)


def _gather(ohT, table, precision=None):
    return lax.dot_general(ohT, table, (((0,), (0,)), ((), ())),
                           preferred_element_type=_f32, precision=precision)


def _gather2(ohT, hi, lo):
    return _gather(ohT, hi) + _gather(ohT, lo)


def _scatter(ohT, vals):
    return jnp.dot(ohT, vals, preferred_element_type=_f32)


def _split(x):
    hi = x.astype(_bf16)
    return hi, (x - hi.astype(_f32)).astype(_bf16)


def _edge_geom(oht_s, oht_d, pos_pad):
    vec = (_gather(oht_d.astype(_f32), pos_pad, _HI)
           - _gather(oht_s.astype(_f32), pos_pad, _HI))
    x, y, z = vec[:, 0:1], vec[:, 1:2], vec[:, 2:3]
    d = jnp.sqrt(x * x + y * y + z * z + 1e-12)
    return vec, d


def _rbf_block(d):
    n = lax.broadcasted_iota(jnp.int32, (1, NB), 1).astype(_f32) + 1.0
    xc = d / CUTOFF
    rbf = jnp.sqrt(2.0 / CUTOFF) * jnp.sin(n * (jnp.pi * xc)) / d
    u = jnp.clip(xc, 0.0, 1.0)
    fc = 1.0 - 10.0 * u ** 3 + 15.0 * u ** 4 - 6.0 * u ** 5
    return rbf * fc


def _rsh16(vec, d):
    u = vec / d
    x, y, z = u[:, 0:1], u[:, 1:2], u[:, 2:3]
    s3 = jnp.sqrt(3.0)
    cols = [jnp.ones_like(x), x, y, z, s3 * x * y, s3 * y * z,
            0.5 * (3.0 * z * z - 1.0), s3 * x * z, 0.5 * s3 * (x * x - y * y)]
    blk = x.shape[0]
    out = jnp.zeros((blk, 16), _f32)
    for k, c in enumerate(cols):
        sel = (lax.broadcasted_iota(jnp.int32, (1, 16), 1) == k).astype(_f32)
        out = out + c * sel
    return out


def _geom_e_body(src_ref, dst_ref, pos_ref, wf_ref, bf_ref, w_ref, rsh_ref):
    oht_s = _onehot_T(src_ref[0], N, BE, _f32)
    oht_d = _onehot_T(dst_ref[0], N, BE, _f32)
    vec, d = _edge_geom(oht_s, oht_d, pos_ref[...])
    rbf = _rbf_block(d)
    w_ref[...] = _silu(_dot_ref(rbf, wf_ref[...]) + bf_ref[...])
    rsh_ref[...] = _rsh16(vec, d)


def _geom_e(src3, dst3, pos_pad, wf_flat, b2d):
    nblk = E // BE
    return pl.pallas_call(
        _geom_e_body,
        grid=(nblk,),
        in_specs=[
            pl.BlockSpec((1, 1, BE), lambda i: (i, 0, 0)),
            pl.BlockSpec((1, 1, BE), lambda i: (i, 0, 0)),
            pl.BlockSpec((N, 8), lambda i: (0, 0)),
            pl.BlockSpec((NB, NL * C), lambda i: (0, 0)),
            pl.BlockSpec((1, NL * C), lambda i: (0, 0)),
        ],
        out_specs=[
            pl.BlockSpec((BE, NL * C), lambda i: (i, 0)),
            pl.BlockSpec((BE, 16), lambda i: (i, 0)),
        ],
        out_shape=[
            jax.ShapeDtypeStruct((E, NL * C), _f32),
            jax.ShapeDtypeStruct((E, 16), _f32),
        ],
    )(src3, dst3, pos_pad, wf_flat, b2d)


def _geom_ef_body(src_ref, dst_ref, pos_ref, s0h_ref, s0l_ref, wc_ref, wp_ref,
                  fr2_ref, frsh_ref, pg_ref):
    oht_s = _onehot_T(src_ref[0], N, BF, _bf16)
    oht_d = _onehot_T(dst_ref[0], N, BF, _bf16)
    vec, d = _edge_geom(oht_s, oht_d, pos_ref[...])
    rbf = _rbf_block(d)
    fr2_ref[...] = _dot_ref(rbf, wc_ref[...])
    frsh_ref[...] = _rsh16(vec, d)
    oht_sum = oht_s + oht_d
    n0sum = _gather2(oht_sum, s0h_ref[...], s0l_ref[...])
    pg_ref[...] = _silu(_dot_ref(n0sum, wp_ref[...]))


def _geom_ef(fsrc3, fdst3, pos_pad, s0_hi, s0_lo, wc_cat, wp):
    nblk = EF // BF
    return pl.pallas_call(
        _geom_ef_body,
        grid=(nblk,),
        in_specs=[
            pl.BlockSpec((1, 1, BF), lambda i: (i, 0, 0)),
            pl.BlockSpec((1, 1, BF), lambda i: (i, 0, 0)),
            pl.BlockSpec((N, 8), lambda i: (0, 0)),
            pl.BlockSpec((N, C), lambda i: (0, 0)),
            pl.BlockSpec((N, C), lambda i: (0, 0)),
            pl.BlockSpec((NB, NA * C), lambda i: (0, 0)),
            pl.BlockSpec((C, 9 * S), lambda i: (0, 0)),
        ],
        out_specs=[
            pl.BlockSpec((BF, NA * C), lambda i: (i, 0)),
            pl.BlockSpec((BF, 16), lambda i: (i, 0)),
            pl.BlockSpec((BF, 9 * S), lambda i: (i, 0)),
        ],
        out_shape=[
            jax.ShapeDtypeStruct((EF, NA * C), _f32),
            jax.ShapeDtypeStruct((EF, 16), _f32),
            jax.ShapeDtypeStruct((EF, 9 * S), _f32),
        ],
    )(fsrc3, fdst3, pos_pad, s0_hi, s0_lo, wc_cat, wp)


def _embed_body(at_ref, tab_ref, s0_ref, hi_ref, lo_ref):
    idx2d = jnp.reshape(at_ref[...], (1, N))
    oht = _onehot_T(idx2d, 128, N, _f32)
    s0 = _gather(oht, tab_ref[...], _HI)
    s0_ref[...] = s0
    hi, lo = _split(s0)
    hi_ref[...] = hi
    lo_ref[...] = lo


def _embed(at_no, embed_pad):
    return pl.pallas_call(
        _embed_body,
        in_specs=[pl.BlockSpec((N,), lambda: (0,)),
                  pl.BlockSpec((128, C), lambda: (0, 0))],
        out_specs=[pl.BlockSpec((N, C), lambda: (0, 0)),
                   pl.BlockSpec((N, C), lambda: (0, 0)),
                   pl.BlockSpec((N, C), lambda: (0, 0))],
        out_shape=[jax.ShapeDtypeStruct((N, C), _f32),
                   jax.ShapeDtypeStruct((N, C), _bf16),
                   jax.ShapeDtypeStruct((N, C), _bf16)],
        grid=(),
    )(at_no, embed_pad)


def _layer_pre_body(use_gate, s_ref, v_ref, ws_ref, wg_ref, hs_ref, lo_ref):
    hs = _dot_ref(s_ref[...], ws_ref[...])
    if use_gate:
        vn2 = jnp.zeros((N, C), _f32)
        for k in range(9):
            vk = v_ref[:, k * C:(k + 1) * C]
            vn2 = vn2 + vk * vk
        vn = jnp.sqrt(vn2 + 1e-6)
        hs = hs * _sigmoid(_dot_ref(vn, wg_ref[...]))
    hi, lo = _split(hs)
    hs_ref[...] = hi
    lo_ref[...] = lo


def _layer_pre(s, vflat, w_self_i, w_gate_i, use_gate):
    return pl.pallas_call(
        functools.partial(_layer_pre_body, use_gate),
        in_specs=[pl.BlockSpec((N, C), lambda: (0, 0)),
                  pl.BlockSpec((N, 9 * C), lambda: (0, 0)),
                  pl.BlockSpec((C, C), lambda: (0, 0)),
                  pl.BlockSpec((C, C), lambda: (0, 0))],
        out_specs=[pl.BlockSpec((N, C), lambda: (0, 0)),
                   pl.BlockSpec((N, C), lambda: (0, 0))],
        out_shape=[jax.ShapeDtypeStruct((N, C), _bf16),
                   jax.ShapeDtypeStruct((N, C), _bf16)],
        grid=(),
    )(s, vflat, w_self_i, w_gate_i)


def _edge_msg_body(do_v, src_ref, dst_ref, w_ref, rsh_ref, hs_ref, hslo_ref,
                   aggs_ref, aggv_ref):
    i = pl.program_id(0)
    oht_s = _onehot_T(src_ref[0], N, BE, _bf16)
    oht_d = _onehot_T(dst_ref[0], N, BE, _bf16)
    hsg = _gather2(oht_s, hs_ref[...], hslo_ref[...])
    m = w_ref[...] * hsg
    m_hi, m_lo = _split(m)

    @pl.when(i == 0)
    def _():
        aggs_ref[...] = jnp.zeros_like(aggs_ref)
        if do_v:
            aggv_ref[...] = jnp.zeros_like(aggv_ref)

    aggs_ref[...] += _scatter(oht_d, m_hi) + _scatter(oht_d, m_lo)
    if do_v:
        rsh = rsh_ref[...]
        mv = jnp.concatenate([m * rsh[:, k:k + 1] for k in range(9)], axis=1)
        mv_hi, mv_lo = _split(mv)
        aggv_ref[...] += _scatter(oht_d, mv_hi) + _scatter(oht_d, mv_lo)


def _edge_msg(src3, dst3, w_all, rsh_e, hs_bf, hs_lo, layer, do_v):
    nblk = E // BE
    return pl.pallas_call(
        functools.partial(_edge_msg_body, do_v),
        grid=(nblk,),
        in_specs=[
            pl.BlockSpec((1, 1, BE), lambda i: (i, 0, 0)),
            pl.BlockSpec((1, 1, BE), lambda i: (i, 0, 0)),
            pl.BlockSpec((BE, C), lambda i, L=layer: (i, L)),
            pl.BlockSpec((BE, 16), lambda i: (i, 0)),
            pl.BlockSpec((N, C), lambda i: (0, 0)),
            pl.BlockSpec((N, C), lambda i: (0, 0)),
        ],
        out_specs=[
            pl.BlockSpec((N, C), lambda i: (0, 0)),
            pl.BlockSpec((N, 9 * C), lambda i: (0, 0)),
        ],
        out_shape=[
            jax.ShapeDtypeStruct((N, C), _f32),
            jax.ShapeDtypeStruct((N, 9 * C), _f32),
        ],
    )(src3, dst3, w_all, rsh_e, hs_bf, hs_lo)


def _layer_post_body(tail, s_ref, v_ref, aggs_ref, aggv_ref, wu1_ref, wu2_ref,
                     wab_ref, wn1_ref, wn2_ref, s_out, v_out,
                     a12h_ref, a12l_ref, hn_ref):
    up = _silu(_dot_ref(aggs_ref[...], wu1_ref[...]))
    s_new = s_ref[...] + _dot_ref(up, wu2_ref[...])
    s_out[...] = s_new
    v_out[...] = v_ref[...] + aggv_ref[...]
    if tail:
        a12 = _dot_ref(s_new, wab_ref[...])
        hi, lo = _split(a12)
        a12h_ref[...] = hi
        a12l_ref[...] = lo
        h1 = _silu(_dot_ref(s_new, wn1_ref[...]))
        hn_ref[...] = _dot_ref(h1, wn2_ref[...])


def _layer_post(s, vflat, agg_s, agg_v, wu1, wu2, wab, wn1, wn2, tail):
    return pl.pallas_call(
        functools.partial(_layer_post_body, tail),
        in_specs=[pl.BlockSpec((N, C), lambda: (0, 0)),
                  pl.BlockSpec((N, 9 * C), lambda: (0, 0)),
                  pl.BlockSpec((N, C), lambda: (0, 0)),
                  pl.BlockSpec((N, 9 * C), lambda: (0, 0)),
                  pl.BlockSpec((C, C), lambda: (0, 0)),
                  pl.BlockSpec((C, C), lambda: (0, 0)),
                  pl.BlockSpec((C, 2 * C), lambda: (0, 0)),
                  pl.BlockSpec((C, H), lambda: (0, 0)),
                  pl.BlockSpec((H, 9 * S), lambda: (0, 0))],
        out_specs=[pl.BlockSpec((N, C), lambda: (0, 0)),
                   pl.BlockSpec((N, 9 * C), lambda: (0, 0)),
                   pl.BlockSpec((N, 2 * C), lambda: (0, 0)),
                   pl.BlockSpec((N, 2 * C), lambda: (0, 0)),
                   pl.BlockSpec((N, 9 * S), lambda: (0, 0))],
        out_shape=[jax.ShapeDtypeStruct((N, C), _f32),
                   jax.ShapeDtypeStruct((N, 9 * C), _f32),
                   jax.ShapeDtypeStruct((N, 2 * C), _bf16),
                   jax.ShapeDtypeStruct((N, 2 * C), _bf16),
                   jax.ShapeDtypeStruct((N, 9 * S), _f32)],
        grid=(),
    )(s, vflat, agg_s, agg_v, wu1, wu2, wab, wn1, wn2)


def _edge_he_body(src_ref, dst_ref, fr2_ref, frsh_ref, pg_ref,
                  ash_ref, asl_ref, adh_ref, adl_ref,
                  we2a_ref, we2b_ref, weo_ref, emat_ref, nacc_ref):
    i = pl.program_id(0)
    oht_s = _onehot_T(src_ref[0], N, BF, _bf16)
    oht_d = _onehot_T(dst_ref[0], N, BF, _bf16)
    gs = _gather2(oht_s, ash_ref[...], asl_ref[...])
    gd = _gather2(oht_d, adh_ref[...], adl_ref[...])
    fr2 = fr2_ref[...]
    g0 = gs[:, :C] + gd[:, :C] + fr2[:, :C]
    g1 = gs[:, C:] + gd[:, C:] + fr2[:, C:]
    he = _dot_ref(_silu(g0), we2a_ref[...]) + _dot_ref(_silu(g1), we2b_ref[...])
    frsh = frsh_ref[...]
    acc = jnp.concatenate(
        [he[:, k * S:(k + 1) * S] * frsh[:, k:k + 1] for k in range(9)], axis=1)
    emat_ref[...] = _dot_ref(acc * pg_ref[...], weo_ref[...])

    @pl.when(i == 0)
    def _():
        nacc_ref[...] = jnp.zeros_like(nacc_ref)

    acc_hi, acc_lo = _split(acc)
    nacc_ref[...] += _scatter(oht_d, acc_hi) + _scatter(oht_d, acc_lo)


def _edge_he(fsrc3, fdst3, fr2, frsh, pg, ash, asl, adh, adl, we2a, we2b, weo):
    nblk = EF // BF
    return pl.pallas_call(
        _edge_he_body,
        grid=(nblk,),
        in_specs=[
            pl.BlockSpec((1, 1, BF), lambda i: (i, 0, 0)),
            pl.BlockSpec((1, 1, BF), lambda i: (i, 0, 0)),
            pl.BlockSpec((BF, NA * C), lambda i: (i, 0)),
            pl.BlockSpec((BF, 16), lambda i: (i, 0)),
            pl.BlockSpec((BF, 9 * S), lambda i: (i, 0)),
            pl.BlockSpec((N, NA * C), lambda i: (0, 0)),
            pl.BlockSpec((N, NA * C), lambda i: (0, 0)),
            pl.BlockSpec((N, NA * C), lambda i: (0, 0)),
            pl.BlockSpec((N, NA * C), lambda i: (0, 0)),
            pl.BlockSpec((C, 9 * S), lambda i: (0, 0)),
            pl.BlockSpec((C, 9 * S), lambda i: (0, 0)),
            pl.BlockSpec((9 * S, B * B), lambda i: (0, 0)),
        ],
        out_specs=[
            pl.BlockSpec((BF, B * B), lambda i: (i, 0)),
            pl.BlockSpec((N, 9 * S), lambda i: (0, 0)),
        ],
        out_shape=[
            jax.ShapeDtypeStruct((EF, B * B), _f32),
            jax.ShapeDtypeStruct((N, 9 * S), _f32),
        ],
    )(fsrc3, fdst3, fr2, frsh, pg, ash, asl, adh, adl, we2a, we2b, weo)


def _node_out_body(s0_ref, hn0_ref, hn1_ref, nacc_ref, wg0_ref, wno_ref, out_ref):
    node_sph = hn0_ref[...] + hn1_ref[...] + nacc_ref[...]
    g0 = _silu(_dot_ref(s0_ref[...], wg0_ref[...]))
    out_ref[...] = _dot_ref(node_sph * g0, wno_ref[...])


def _node_out(s0, hn0, hn1, nacc, wg0, wno):
    return pl.pallas_call(
        _node_out_body,
        in_specs=[pl.BlockSpec((N, C), lambda: (0, 0)),
                  pl.BlockSpec((N, 9 * S), lambda: (0, 0)),
                  pl.BlockSpec((N, 9 * S), lambda: (0, 0)),
                  pl.BlockSpec((N, 9 * S), lambda: (0, 0)),
                  pl.BlockSpec((C, 9 * S), lambda: (0, 0)),
                  pl.BlockSpec((9 * S, B * B), lambda: (0, 0))],
        out_specs=pl.BlockSpec((N, B * B), lambda: (0, 0)),
        out_shape=jax.ShapeDtypeStruct((N, B * B), _f32),
        grid=(),
    )(s0, hn0, hn1, nacc, wg0, wno)


def kernel(at_no, pos, edge_index, fc_edge_index, embed_table, W_filt, b_filt,
           W_self, W_gate, W_up1, W_up2, Wn1, Wn2, We1, We2, Wg0, Wnode_out,
           Wp, Wedge_out):
    src3 = edge_index[0].reshape(E // BE, 1, BE).astype(jnp.int32)
    dst3 = edge_index[1].reshape(E // BE, 1, BE).astype(jnp.int32)
    fsrc3 = fc_edge_index[0].reshape(EF // BF, 1, BF).astype(jnp.int32)
    fdst3 = fc_edge_index[1].reshape(EF // BF, 1, BF).astype(jnp.int32)
    pos_pad = jnp.zeros((N, 8), _f32).at[:, :3].set(pos)
    embed_pad = jnp.zeros((128, C), _f32).at[:100].set(embed_table)
    wf_flat = jnp.transpose(W_filt, (1, 0, 2)).reshape(NB, NL * C)
    b2d = b_filt.reshape(1, NL * C)
    wc_cat = jnp.transpose(We1[:, 2 * C:, :], (1, 0, 2)).reshape(NB, NA * C)

    s0, s0_hi, s0_lo = _embed(at_no.astype(jnp.int32), embed_pad)

    w_all, rsh_e = _geom_e(src3, dst3, pos_pad, wf_flat, b2d)
    fr2, frsh, pg = _geom_ef(fsrc3, fdst3, pos_pad, s0_hi, s0_lo, wc_cat, Wp)

    s = s0
    vflat = jnp.zeros((N, 9 * C), _f32)
    a12h, a12l, hn = [], [], []
    for idx in range(NL):
        hs_bf, hs_lo = _layer_pre(s, vflat, W_self[idx], W_gate[idx],
                                  use_gate=idx > 0)
        agg_s, agg_v = _edge_msg(src3, dst3, w_all, rsh_e, hs_bf, hs_lo, idx,
                                 do_v=idx < NL - 1)
        tail = idx >= NL - NA
        j = idx - (NL - NA)
        wab = (jnp.concatenate([We1[j, :C, :], We1[j, C:2 * C, :]], axis=1)
               if tail else jnp.zeros((C, 2 * C), _f32))
        s, vflat, a12h_i, a12l_i, hn_i = _layer_post(
            s, vflat, agg_s, agg_v, W_up1[idx], W_up2[idx], wab,
            Wn1[j] if tail else jnp.zeros((C, H), _f32),
            Wn2[j] if tail else jnp.zeros((H, 9 * S), _f32), tail)
        if tail:
            a12h.append(a12h_i)
            a12l.append(a12l_i)
            hn.append(hn_i)

    ash = jnp.concatenate([a12h[0][:, :C], a12h[1][:, :C]], axis=1)
    asl = jnp.concatenate([a12l[0][:, :C], a12l[1][:, :C]], axis=1)
    adh = jnp.concatenate([a12h[0][:, C:], a12h[1][:, C:]], axis=1)
    adl = jnp.concatenate([a12l[0][:, C:], a12l[1][:, C:]], axis=1)
    emat, nacc = _edge_he(fsrc3, fdst3, fr2, frsh, pg, ash, asl, adh, adl,
                          We2[0], We2[1], Wedge_out)
    nmat = _node_out(s0, hn[0], hn[1], nacc, Wg0, Wnode_out)
    return nmat.reshape(N, B, B), emat.reshape(EF, B, B)


# speedup vs baseline: 12.2932x; 1.1613x over previous
import functools
import jax
import jax.numpy as jnp
from jax import lax
from jax.experimental import pallas as pl

N = 1024
E = 16384
EF = 65536
C = 128
NB = 32
H = 128
S = 32
B = 16
NL = 4
NA = 2
CUTOFF = 5.0

BE = 2048
BF = 2048

_bf16 = jnp.bfloat16
_f32 = jnp.float32
_HI = lax.Precision.HIGHEST


def _sigmoid(x):
    return 1.0 / (1.0 + jnp.exp(-x))


def _silu(x):
    return x * _sigmoid(x)


def _dot_ref(a, b):
    return jnp.dot(a.astype(_bf16), b.astype(_bf16), preferred_element_type=_f32)


def _onehot_T(idx2d, rows, cols, dtype):
    return (lax.broadcasted_iota(jnp.int32, (rows, cols), 0) == idx2d).astype(dtype)


def _gather(ohT, table, precision=None):
    return lax.dot_general(ohT, table, (((0,), (0,)), ((), ())),
                           preferred_element_type=_f32, precision=precision)


def _gather2(ohT, hi, lo):
    return _gather(ohT, hi) + _gather(ohT, lo)


def _scatter(ohT, vals):
    return jnp.dot(ohT, vals, preferred_element_type=_f32)


def _split(x):
    hi = x.astype(_bf16)
    return hi, (x - hi.astype(_f32)).astype(_bf16)


def _edge_geom(oht_s, oht_d, pos_pad):
    vec = (_gather(oht_d.astype(_f32), pos_pad, _HI)
           - _gather(oht_s.astype(_f32), pos_pad, _HI))
    x, y, z = vec[:, 0:1], vec[:, 1:2], vec[:, 2:3]
    d = jnp.sqrt(x * x + y * y + z * z + 1e-12)
    return vec, d


def _rbf_block(d):
    n = lax.broadcasted_iota(jnp.int32, (1, NB), 1).astype(_f32) + 1.0
    xc = d / CUTOFF
    rbf = jnp.sqrt(2.0 / CUTOFF) * jnp.sin(n * (jnp.pi * xc)) / d
    u = jnp.clip(xc, 0.0, 1.0)
    fc = 1.0 - 10.0 * u ** 3 + 15.0 * u ** 4 - 6.0 * u ** 5
    return rbf * fc


def _rsh16(vec, d):
    u = vec / d
    x, y, z = u[:, 0:1], u[:, 1:2], u[:, 2:3]
    s3 = jnp.sqrt(3.0)
    cols = [jnp.ones_like(x), x, y, z, s3 * x * y, s3 * y * z,
            0.5 * (3.0 * z * z - 1.0), s3 * x * z, 0.5 * s3 * (x * x - y * y)]
    blk = x.shape[0]
    out = jnp.zeros((blk, 16), _f32)
    for k, c in enumerate(cols):
        sel = (lax.broadcasted_iota(jnp.int32, (1, 16), 1) == k).astype(_f32)
        out = out + c * sel
    return out


def _geom_e_body(src_ref, dst_ref, pos_ref, wf_ref, bf_ref, w_ref, rsh_ref):
    oht_s = _onehot_T(src_ref[0], N, BE, _f32)
    oht_d = _onehot_T(dst_ref[0], N, BE, _f32)
    vec, d = _edge_geom(oht_s, oht_d, pos_ref[...])
    rbf = _rbf_block(d)
    w_ref[...] = _silu(_dot_ref(rbf, wf_ref[...]) + bf_ref[...])
    rsh_ref[...] = _rsh16(vec, d)


def _geom_e(src3, dst3, pos_pad, wf_flat, b2d):
    nblk = E // BE
    return pl.pallas_call(
        _geom_e_body,
        grid=(nblk,),
        in_specs=[
            pl.BlockSpec((1, 1, BE), lambda i: (i, 0, 0)),
            pl.BlockSpec((1, 1, BE), lambda i: (i, 0, 0)),
            pl.BlockSpec((N, 8), lambda i: (0, 0)),
            pl.BlockSpec((NB, NL * C), lambda i: (0, 0)),
            pl.BlockSpec((1, NL * C), lambda i: (0, 0)),
        ],
        out_specs=[
            pl.BlockSpec((BE, NL * C), lambda i: (i, 0)),
            pl.BlockSpec((BE, 16), lambda i: (i, 0)),
        ],
        out_shape=[
            jax.ShapeDtypeStruct((E, NL * C), _f32),
            jax.ShapeDtypeStruct((E, 16), _f32),
        ],
    )(src3, dst3, pos_pad, wf_flat, b2d)


def _geom_ef_body(src_ref, dst_ref, pos_ref, s0h_ref, wc_ref, wp_ref,
                  fr2_ref, frsh_ref, pg_ref):
    oht_s = _onehot_T(src_ref[0], N, BF, _bf16)
    oht_d = _onehot_T(dst_ref[0], N, BF, _bf16)
    vec, d = _edge_geom(oht_s, oht_d, pos_ref[...])
    rbf = _rbf_block(d)
    fr2_ref[...] = _dot_ref(rbf, wc_ref[...])
    frsh_ref[...] = _rsh16(vec, d)
    n0sum = _gather(oht_s + oht_d, s0h_ref[...])
    pg_ref[...] = _silu(_dot_ref(n0sum, wp_ref[...]))


def _geom_ef(fsrc3, fdst3, pos_pad, s0_hi, wc_cat, wp):
    nblk = EF // BF
    return pl.pallas_call(
        _geom_ef_body,
        grid=(nblk,),
        in_specs=[
            pl.BlockSpec((1, 1, BF), lambda i: (i, 0, 0)),
            pl.BlockSpec((1, 1, BF), lambda i: (i, 0, 0)),
            pl.BlockSpec((N, 8), lambda i: (0, 0)),
            pl.BlockSpec((N, C), lambda i: (0, 0)),
            pl.BlockSpec((NB, NA * C), lambda i: (0, 0)),
            pl.BlockSpec((C, 9 * S), lambda i: (0, 0)),
        ],
        out_specs=[
            pl.BlockSpec((BF, NA * C), lambda i: (i, 0)),
            pl.BlockSpec((BF, 16), lambda i: (i, 0)),
            pl.BlockSpec((BF, 9 * S), lambda i: (i, 0)),
        ],
        out_shape=[
            jax.ShapeDtypeStruct((EF, NA * C), _f32),
            jax.ShapeDtypeStruct((EF, 16), _f32),
            jax.ShapeDtypeStruct((EF, 9 * S), _f32),
        ],
    )(fsrc3, fdst3, pos_pad, s0_hi, wc_cat, wp)


def _embed_body(at_ref, tab_ref, s0_ref, hi_ref, lo_ref):
    idx2d = jnp.reshape(at_ref[...], (1, N))
    oht = _onehot_T(idx2d, 128, N, _f32)
    s0 = _gather(oht, tab_ref[...], _HI)
    s0_ref[...] = s0
    hi, lo = _split(s0)
    hi_ref[...] = hi
    lo_ref[...] = lo


def _embed(at_no, embed_pad):
    return pl.pallas_call(
        _embed_body,
        in_specs=[pl.BlockSpec((N,), lambda: (0,)),
                  pl.BlockSpec((128, C), lambda: (0, 0))],
        out_specs=[pl.BlockSpec((N, C), lambda: (0, 0)),
                   pl.BlockSpec((N, C), lambda: (0, 0)),
                   pl.BlockSpec((N, C), lambda: (0, 0))],
        out_shape=[jax.ShapeDtypeStruct((N, C), _f32),
                   jax.ShapeDtypeStruct((N, C), _bf16),
                   jax.ShapeDtypeStruct((N, C), _bf16)],
        grid=(),
    )(at_no, embed_pad)


def _layer_pre_body(use_gate, s_ref, v_ref, ws_ref, wg_ref, hs_ref, lo_ref):
    hs = _dot_ref(s_ref[...], ws_ref[...])
    if use_gate:
        vn2 = jnp.zeros((N, C), _f32)
        for k in range(9):
            vk = v_ref[:, k * C:(k + 1) * C]
            vn2 = vn2 + vk * vk
        vn = jnp.sqrt(vn2 + 1e-6)
        hs = hs * _sigmoid(_dot_ref(vn, wg_ref[...]))
    hi, lo = _split(hs)
    hs_ref[...] = hi
    lo_ref[...] = lo


def _layer_pre(s, vflat, w_self_i, w_gate_i, use_gate):
    return pl.pallas_call(
        functools.partial(_layer_pre_body, use_gate),
        in_specs=[pl.BlockSpec((N, C), lambda: (0, 0)),
                  pl.BlockSpec((N, 9 * C), lambda: (0, 0)),
                  pl.BlockSpec((C, C), lambda: (0, 0)),
                  pl.BlockSpec((C, C), lambda: (0, 0))],
        out_specs=[pl.BlockSpec((N, C), lambda: (0, 0)),
                   pl.BlockSpec((N, C), lambda: (0, 0))],
        out_shape=[jax.ShapeDtypeStruct((N, C), _bf16),
                   jax.ShapeDtypeStruct((N, C), _bf16)],
        grid=(),
    )(s, vflat, w_self_i, w_gate_i)


def _edge_msg_body(do_v, src_ref, dst_ref, w_ref, rsh_ref, hs_ref, hslo_ref,
                   aggs_ref, aggv_ref):
    i = pl.program_id(0)
    oht_s = _onehot_T(src_ref[0], N, BE, _bf16)
    oht_d = _onehot_T(dst_ref[0], N, BE, _bf16)
    hsg = _gather2(oht_s, hs_ref[...], hslo_ref[...])
    m = w_ref[...] * hsg
    m_hi, m_lo = _split(m)

    @pl.when(i == 0)
    def _():
        aggs_ref[...] = jnp.zeros_like(aggs_ref)
        if do_v:
            aggv_ref[...] = jnp.zeros_like(aggv_ref)

    aggs_ref[...] += _scatter(oht_d, m_hi) + _scatter(oht_d, m_lo)
    if do_v:
        rsh = rsh_ref[...]
        mv = jnp.concatenate([m * rsh[:, k:k + 1] for k in range(9)], axis=1)
        mv_hi, mv_lo = _split(mv)
        aggv_ref[...] += _scatter(oht_d, mv_hi) + _scatter(oht_d, mv_lo)


def _edge_msg(src3, dst3, w_all, rsh_e, hs_bf, hs_lo, layer, do_v):
    nblk = E // BE
    return pl.pallas_call(
        functools.partial(_edge_msg_body, do_v),
        grid=(nblk,),
        in_specs=[
            pl.BlockSpec((1, 1, BE), lambda i: (i, 0, 0)),
            pl.BlockSpec((1, 1, BE), lambda i: (i, 0, 0)),
            pl.BlockSpec((BE, C), lambda i, L=layer: (i, L)),
            pl.BlockSpec((BE, 16), lambda i: (i, 0)),
            pl.BlockSpec((N, C), lambda i: (0, 0)),
            pl.BlockSpec((N, C), lambda i: (0, 0)),
        ],
        out_specs=[
            pl.BlockSpec((N, C), lambda i: (0, 0)),
            pl.BlockSpec((N, 9 * C), lambda i: (0, 0)),
        ],
        out_shape=[
            jax.ShapeDtypeStruct((N, C), _f32),
            jax.ShapeDtypeStruct((N, 9 * C), _f32),
        ],
    )(src3, dst3, w_all, rsh_e, hs_bf, hs_lo)


def _layer_post_body(tail, s_ref, v_ref, aggs_ref, aggv_ref, wu1_ref, wu2_ref,
                     wab_ref, wn1_ref, wn2_ref, s_out, v_out,
                     a12h_ref, a12l_ref, hn_ref):
    up = _silu(_dot_ref(aggs_ref[...], wu1_ref[...]))
    s_new = s_ref[...] + _dot_ref(up, wu2_ref[...])
    s_out[...] = s_new
    v_out[...] = v_ref[...] + aggv_ref[...]
    if tail:
        a12 = _dot_ref(s_new, wab_ref[...])
        hi, lo = _split(a12)
        a12h_ref[...] = hi
        a12l_ref[...] = lo
        h1 = _silu(_dot_ref(s_new, wn1_ref[...]))
        hn_ref[...] = _dot_ref(h1, wn2_ref[...])


def _layer_post(s, vflat, agg_s, agg_v, wu1, wu2, wab, wn1, wn2, tail):
    return pl.pallas_call(
        functools.partial(_layer_post_body, tail),
        in_specs=[pl.BlockSpec((N, C), lambda: (0, 0)),
                  pl.BlockSpec((N, 9 * C), lambda: (0, 0)),
                  pl.BlockSpec((N, C), lambda: (0, 0)),
                  pl.BlockSpec((N, 9 * C), lambda: (0, 0)),
                  pl.BlockSpec((C, C), lambda: (0, 0)),
                  pl.BlockSpec((C, C), lambda: (0, 0)),
                  pl.BlockSpec((C, 2 * C), lambda: (0, 0)),
                  pl.BlockSpec((C, H), lambda: (0, 0)),
                  pl.BlockSpec((H, 9 * S), lambda: (0, 0))],
        out_specs=[pl.BlockSpec((N, C), lambda: (0, 0)),
                   pl.BlockSpec((N, 9 * C), lambda: (0, 0)),
                   pl.BlockSpec((N, 2 * C), lambda: (0, 0)),
                   pl.BlockSpec((N, 2 * C), lambda: (0, 0)),
                   pl.BlockSpec((N, 9 * S), lambda: (0, 0))],
        out_shape=[jax.ShapeDtypeStruct((N, C), _f32),
                   jax.ShapeDtypeStruct((N, 9 * C), _f32),
                   jax.ShapeDtypeStruct((N, 2 * C), _bf16),
                   jax.ShapeDtypeStruct((N, 2 * C), _bf16),
                   jax.ShapeDtypeStruct((N, 9 * S), _f32)],
        grid=(),
    )(s, vflat, agg_s, agg_v, wu1, wu2, wab, wn1, wn2)


def _edge_he_body(src_ref, dst_ref, fr2_ref, frsh_ref, pg_ref,
                  ash_ref, adh_ref,
                  we2a_ref, we2b_ref, weo_ref, emat_ref, nacc_ref):
    i = pl.program_id(0)
    oht_s = _onehot_T(src_ref[0], N, BF, _bf16)
    oht_d = _onehot_T(dst_ref[0], N, BF, _bf16)
    gs = _gather(oht_s, ash_ref[...])
    gd = _gather(oht_d, adh_ref[...])
    fr2 = fr2_ref[...]
    g0 = gs[:, :C] + gd[:, :C] + fr2[:, :C]
    g1 = gs[:, C:] + gd[:, C:] + fr2[:, C:]
    he = _dot_ref(_silu(g0), we2a_ref[...]) + _dot_ref(_silu(g1), we2b_ref[...])
    frsh = frsh_ref[...]
    acc = jnp.concatenate(
        [he[:, k * S:(k + 1) * S] * frsh[:, k:k + 1] for k in range(9)], axis=1)
    emat_ref[...] = _dot_ref(acc * pg_ref[...], weo_ref[...])

    @pl.when(i == 0)
    def _():
        nacc_ref[...] = jnp.zeros_like(nacc_ref)

    nacc_ref[...] += _scatter(oht_d, acc.astype(_bf16))


def _edge_he(fsrc3, fdst3, fr2, frsh, pg, ash, adh, we2a, we2b, weo):
    nblk = EF // BF
    return pl.pallas_call(
        _edge_he_body,
        grid=(nblk,),
        in_specs=[
            pl.BlockSpec((1, 1, BF), lambda i: (i, 0, 0)),
            pl.BlockSpec((1, 1, BF), lambda i: (i, 0, 0)),
            pl.BlockSpec((BF, NA * C), lambda i: (i, 0)),
            pl.BlockSpec((BF, 16), lambda i: (i, 0)),
            pl.BlockSpec((BF, 9 * S), lambda i: (i, 0)),
            pl.BlockSpec((N, NA * C), lambda i: (0, 0)),
            pl.BlockSpec((N, NA * C), lambda i: (0, 0)),
            pl.BlockSpec((C, 9 * S), lambda i: (0, 0)),
            pl.BlockSpec((C, 9 * S), lambda i: (0, 0)),
            pl.BlockSpec((9 * S, B * B), lambda i: (0, 0)),
        ],
        out_specs=[
            pl.BlockSpec((BF, B * B), lambda i: (i, 0)),
            pl.BlockSpec((N, 9 * S), lambda i: (0, 0)),
        ],
        out_shape=[
            jax.ShapeDtypeStruct((EF, B * B), _f32),
            jax.ShapeDtypeStruct((N, 9 * S), _f32),
        ],
    )(fsrc3, fdst3, fr2, frsh, pg, ash, adh, we2a, we2b, weo)


def _node_out_body(s0_ref, hn0_ref, hn1_ref, nacc_ref, wg0_ref, wno_ref, out_ref):
    node_sph = hn0_ref[...] + hn1_ref[...] + nacc_ref[...]
    g0 = _silu(_dot_ref(s0_ref[...], wg0_ref[...]))
    out_ref[...] = _dot_ref(node_sph * g0, wno_ref[...])


def _node_out(s0, hn0, hn1, nacc, wg0, wno):
    return pl.pallas_call(
        _node_out_body,
        in_specs=[pl.BlockSpec((N, C), lambda: (0, 0)),
                  pl.BlockSpec((N, 9 * S), lambda: (0, 0)),
                  pl.BlockSpec((N, 9 * S), lambda: (0, 0)),
                  pl.BlockSpec((N, 9 * S), lambda: (0, 0)),
                  pl.BlockSpec((C, 9 * S), lambda: (0, 0)),
                  pl.BlockSpec((9 * S, B * B), lambda: (0, 0))],
        out_specs=pl.BlockSpec((N, B * B), lambda: (0, 0)),
        out_shape=jax.ShapeDtypeStruct((N, B * B), _f32),
        grid=(),
    )(s0, hn0, hn1, nacc, wg0, wno)


def kernel(at_no, pos, edge_index, fc_edge_index, embed_table, W_filt, b_filt,
           W_self, W_gate, W_up1, W_up2, Wn1, Wn2, We1, We2, Wg0, Wnode_out,
           Wp, Wedge_out):
    src3 = edge_index[0].reshape(E // BE, 1, BE).astype(jnp.int32)
    dst3 = edge_index[1].reshape(E // BE, 1, BE).astype(jnp.int32)
    fsrc3 = fc_edge_index[0].reshape(EF // BF, 1, BF).astype(jnp.int32)
    fdst3 = fc_edge_index[1].reshape(EF // BF, 1, BF).astype(jnp.int32)
    pos_pad = jnp.zeros((N, 8), _f32).at[:, :3].set(pos)
    embed_pad = jnp.zeros((128, C), _f32).at[:100].set(embed_table)
    wf_flat = jnp.transpose(W_filt, (1, 0, 2)).reshape(NB, NL * C)
    b2d = b_filt.reshape(1, NL * C)
    wc_cat = jnp.transpose(We1[:, 2 * C:, :], (1, 0, 2)).reshape(NB, NA * C)

    s0, s0_hi, s0_lo = _embed(at_no.astype(jnp.int32), embed_pad)

    w_all, rsh_e = _geom_e(src3, dst3, pos_pad, wf_flat, b2d)
    del s0_lo
    fr2, frsh, pg = _geom_ef(fsrc3, fdst3, pos_pad, s0_hi, wc_cat, Wp)

    s = s0
    vflat = jnp.zeros((N, 9 * C), _f32)
    a12h, a12l, hn = [], [], []
    for idx in range(NL):
        hs_bf, hs_lo = _layer_pre(s, vflat, W_self[idx], W_gate[idx],
                                  use_gate=idx > 0)
        agg_s, agg_v = _edge_msg(src3, dst3, w_all, rsh_e, hs_bf, hs_lo, idx,
                                 do_v=idx < NL - 1)
        tail = idx >= NL - NA
        j = idx - (NL - NA)
        wab = (jnp.concatenate([We1[j, :C, :], We1[j, C:2 * C, :]], axis=1)
               if tail else jnp.zeros((C, 2 * C), _f32))
        s, vflat, a12h_i, a12l_i, hn_i = _layer_post(
            s, vflat, agg_s, agg_v, W_up1[idx], W_up2[idx], wab,
            Wn1[j] if tail else jnp.zeros((C, H), _f32),
            Wn2[j] if tail else jnp.zeros((H, 9 * S), _f32), tail)
        if tail:
            a12h.append(a12h_i)
            a12l.append(a12l_i)
            hn.append(hn_i)

    del a12l
    ash = jnp.concatenate([a12h[0][:, :C], a12h[1][:, :C]], axis=1)
    adh = jnp.concatenate([a12h[0][:, C:], a12h[1][:, C:]], axis=1)
    emat, nacc = _edge_he(fsrc3, fdst3, fr2, frsh, pg, ash, adh,
                          We2[0], We2[1], Wedge_out)
    nmat = _node_out(s0, hn[0], hn[1], nacc, Wg0, Wnode_out)
    return nmat.reshape(N, B, B), emat.reshape(EF, B, B)
